# baseline (speedup 1.0000x reference)
import functools

import jax
import jax.numpy as jnp
from jax import lax
from jax.experimental import pallas as pl
from jax.experimental.pallas import tpu as pltpu
from jax.experimental.pallas import tpu_sc as plsc

N = 10000
NPAD = 10240
D = 128
E = 320000
K = 64
NCHUNK = 160
PERW = NCHUNK * K
EPAD = 32 * PERW
ROWS_PER_SUB = NPAD // 16
IBUF = 4
NBUF = 2


def _zn_body(x_ref, wt_ref, b_ref, o_ref):
    h = jnp.dot(x_ref[...], wt_ref[...], preferred_element_type=jnp.float32)
    o_ref[...] = jnp.maximum(h + b_ref[...], 0.0)


def _zn_call(xp, WlT, b_l):
    return pl.pallas_call(
        _zn_body,
        grid=(NPAD // 1024,),
        in_specs=[
            pl.BlockSpec((1024, D), lambda i: (i, 0)),
            pl.BlockSpec((D, D), lambda i: (0, 0)),
            pl.BlockSpec((1, D), lambda i: (0, 0)),
        ],
        out_specs=pl.BlockSpec((1024, D), lambda i: (i, 0)),
        out_shape=jax.ShapeDtypeStruct((NPAD, D), jnp.float32),
    )(xp, WlT, b_l)


_mesh = plsc.VectorSubcoreMesh(core_axis_name="c", subcore_axis_name="s")


@functools.partial(
    pl.kernel,
    out_type=(
        jax.ShapeDtypeStruct((2, NPAD, D), jnp.float32),
        jax.ShapeDtypeStruct((2, NPAD), jnp.float32),
    ),
    mesh=_mesh,
    scratch_types=[
        pltpu.VMEM((IBUF, K), jnp.int32),
        pltpu.VMEM((IBUF, K), jnp.int32),
        pltpu.VMEM((IBUF, K), jnp.float32),
        pltpu.VMEM((NBUF, K, 16), jnp.float32),
        pltpu.VMEM((NBUF, K, D), jnp.float32),
        pltpu.VMEM((ROWS_PER_SUB,), jnp.float32),
        pltpu.VMEM_SHARED((NPAD, D), jnp.float32),
        pltpu.VMEM_SHARED((NPAD,), jnp.float32),
        pltpu.SemaphoreType.DMA,
        pltpu.SemaphoreType.DMA,
        pltpu.SemaphoreType.DMA,
        pltpu.SemaphoreType.DMA,
        pltpu.SemaphoreType.DMA,
        pltpu.SemaphoreType.DMA,
        pltpu.SemaphoreType.DMA,
        pltpu.SemaphoreType.DMA,
        pltpu.SemaphoreType.DMA,
        pltpu.SemaphoreType.DMA,
    ],
)
def _sc_agg(zn_hbm, src_hbm, dst_hbm, w_hbm, w16_hbm, p_hbm, pw_hbm,
            isrc, idst, wbuf, w16, rows, wz, acc, accw,
            sm0, sm1, sm2, sm3, sg0, sg1, ss0, ss1, sw0, sw1):
    cid = lax.axis_index("c")
    sid = lax.axis_index("s")
    wid = cid * 16 + sid
    sem_m = [sm0, sm1, sm2, sm3]
    sem_g = [sg0, sg1]
    sem_s = [ss0, ss1]
    sem_w = [sw0, sw1]
    zv = jnp.zeros((16,), jnp.float32)

    _scope = jax.named_scope("scagg_zero")
    _scope.__enter__()

    def _zrow(k, carry):
        for j in range(D // 16):
            rows[0, k, pl.ds(j * 16, 16)] = zv
        return carry

    lax.fori_loop(0, K, _zrow, 0)

    def _zwz(k, carry):
        wz[pl.ds(k * 16, 16)] = zv
        return carry

    lax.fori_loop(0, ROWS_PER_SUB // 16, _zwz, 0)

    row0 = sid * ROWS_PER_SUB
    for t in range(ROWS_PER_SUB // K):
        pltpu.sync_copy(rows.at[0], acc.at[pl.ds(row0 + t * K, K)])
    pltpu.sync_copy(wz, accw.at[pl.ds(row0, ROWS_PER_SUB)])
    plsc.subcore_barrier()
    _scope.__exit__(None, None, None)
    _scope2 = jax.named_scope("scagg_mainloop")
    _scope2.__enter__()

    base = wid * PERW

    def _fire_meta(c, slot):
        off = base + c * K
        pltpu.async_copy(src_hbm.at[pl.ds(off, K)], isrc.at[slot], sem_m[slot])
        pltpu.async_copy(dst_hbm.at[pl.ds(off, K)], idst.at[slot], sem_m[slot])
        pltpu.async_copy(w_hbm.at[pl.ds(off, K)], wbuf.at[slot], sem_m[slot])

    def _wait_meta(slot):
        pltpu.make_async_copy(src_hbm.at[pl.ds(0, K)], isrc.at[slot],
                              sem_m[slot]).wait()
        pltpu.make_async_copy(dst_hbm.at[pl.ds(0, K)], idst.at[slot],
                              sem_m[slot]).wait()
        pltpu.make_async_copy(w_hbm.at[pl.ds(0, K)], wbuf.at[slot],
                              sem_m[slot]).wait()

    def _fire_w16(c, slot):
        off = base + c * K
        pltpu.async_copy(w16_hbm.at[pl.ds(off, K)], w16.at[slot], sem_w[slot])

    def _wait_w16(slot):
        pltpu.make_async_copy(w16_hbm.at[pl.ds(0, K)], w16.at[slot],
                              sem_w[slot]).wait()

    for b in range(3):
        _fire_meta(b, b)
    for b in range(NBUF):
        _fire_w16(b, b)
    _wait_meta(0)
    pltpu.async_copy(zn_hbm.at[isrc.at[0]], rows.at[0], sem_g[0])

    def _chunk4(c4, carry):
        for u in range(4):
            c = c4 * 4 + u
            b = u % NBUF
            bn = (u + 1) % NBUF
            slot = u

            @pl.when(c >= 1)
            def _():
                pltpu.make_async_copy(zn_hbm.at[pl.ds(0, K)], rows.at[bn],
                                      sem_s[bn]).wait()
                pltpu.make_async_copy(w_hbm.at[pl.ds(0, K)],
                                      wbuf.at[(u + 3) % IBUF],
                                      sem_s[bn]).wait()

            @pl.when(c + 3 < NCHUNK)
            def _():
                _fire_meta(c + 3, (u + 3) % IBUF)

            @pl.when(c + 1 < NCHUNK)
            def _():
                sn = (u + 1) % IBUF
                _wait_meta(sn)
                pltpu.async_copy(zn_hbm.at[isrc.at[sn]], rows.at[bn],
                                 sem_g[bn])

            pltpu.make_async_copy(zn_hbm.at[pl.ds(0, K)], rows.at[b],
                                  sem_g[b]).wait()
            _wait_w16(b)

            def _scale(k8, inner, _b=b):
                for dk in range(8):
                    k = k8 * 8 + dk
                    wv = w16[_b, k]
                    for j in range(D // 16):
                        sl = pl.ds(j * 16, 16)
                        rows[_b, k, sl] = rows[_b, k, sl] * wv
                return inner

            lax.fori_loop(0, K // 8, _scale, 0)

            @pl.when(c + 2 < NCHUNK)
            def _():
                _fire_w16(c + 2, b)

            pltpu.async_copy(rows.at[b], acc.at[idst.at[slot]], sem_s[b],
                             add=True)
            pltpu.async_copy(wbuf.at[slot], accw.at[idst.at[slot]],
                             sem_s[b], add=True)
        return carry

    lax.fori_loop(0, NCHUNK // 4, _chunk4, 0)

    bl = (NCHUNK - 1) % NBUF
    sl = (NCHUNK - 1) % IBUF
    pltpu.make_async_copy(zn_hbm.at[pl.ds(0, K)], rows.at[bl], sem_s[bl]).wait()
    pltpu.make_async_copy(w_hbm.at[pl.ds(0, K)], wbuf.at[sl], sem_s[bl]).wait()
    plsc.subcore_barrier()
    _scope2.__exit__(None, None, None)
    _scope3 = jax.named_scope("scagg_writeback")
    _scope3.__enter__()

    pltpu.sync_copy(acc.at[pl.ds(row0, ROWS_PER_SUB)],
                    p_hbm.at[cid, pl.ds(row0, ROWS_PER_SUB)])
    pltpu.sync_copy(accw.at[pl.ds(row0, ROWS_PER_SUB)],
                    pw_hbm.at[cid, pl.ds(row0, ROWS_PER_SUB)])
    _scope3.__exit__(None, None, None)


def _out_body(x_ref, p_ref, pw_ref, wt_ref, b_ref, o_ref):
    aggw = pw_ref[0] + pw_ref[1] + 1.0
    agg = (p_ref[0] + p_ref[1]) / aggw
    h = jnp.dot(x_ref[...], wt_ref[0:D, :], preferred_element_type=jnp.float32)
    h = h + jnp.dot(agg, wt_ref[D:2 * D, :], preferred_element_type=jnp.float32)
    h = jnp.maximum(h + b_ref[...], 0.0)
    nrm = jnp.sqrt(jnp.sum(h * h, axis=1, keepdims=True))
    o_ref[...] = h / jnp.maximum(nrm, 1e-12)


def _out_call(xp, P, Pw_col, WrT, b_r):
    return pl.pallas_call(
        _out_body,
        grid=(NPAD // 1024,),
        in_specs=[
            pl.BlockSpec((1024, D), lambda i: (i, 0)),
            pl.BlockSpec((2, 1024, D), lambda i: (0, i, 0)),
            pl.BlockSpec((2, 1024, 1), lambda i: (0, i, 0)),
            pl.BlockSpec((2 * D, D), lambda i: (0, 0)),
            pl.BlockSpec((1, D), lambda i: (0, 0)),
        ],
        out_specs=pl.BlockSpec((1024, D), lambda i: (i, 0)),
        out_shape=jax.ShapeDtypeStruct((NPAD, D), jnp.float32),
    )(xp, P, Pw_col, WrT, b_r)


def kernel(x, edge_index, w, W_l, b_l, W_r, b_r):
    xp = jnp.pad(x, ((0, NPAD - N), (0, 0)))
    src = jnp.pad(edge_index[0, :], (0, EPAD - E))
    dst = jnp.pad(edge_index[1, :], (0, EPAD - E), constant_values=NPAD - 1)
    wp = jnp.pad(w, (0, EPAD - E))
    w16 = jnp.broadcast_to(wp[:, None], (EPAD, 16))

    zn = _zn_call(xp, W_l.T, b_l.reshape(1, D))
    P, Pw = _sc_agg(zn, src, dst, wp, w16)
    out = _out_call(xp, P, Pw.reshape(2, NPAD, 1), W_r.T, b_r.reshape(1, D))
    return out[:N]


# speedup vs baseline: 1.0485x; 1.0485x over previous
import functools

import jax
import jax.numpy as jnp
from jax import lax
from jax.experimental import pallas as pl
from jax.experimental.pallas import tpu as pltpu
from jax.experimental.pallas import tpu_sc as plsc

N = 10000
NPAD = 10240
D = 128
E = 320000
EPAD = 323584
PERW = EPAD // 32
K = 128
NCHUNK = PERW // K
ROWS_PER_SUB = NPAD // 16


def _zn_body(x_ref, wt_ref, b_ref, o_ref):
    h = jnp.dot(x_ref[...], wt_ref[...], preferred_element_type=jnp.float32)
    o_ref[...] = jnp.maximum(h + b_ref[...], 0.0)


def _zn_call(xp, WlT, b_l):
    return pl.pallas_call(
        _zn_body,
        grid=(NPAD // 1024,),
        in_specs=[
            pl.BlockSpec((1024, D), lambda i: (i, 0)),
            pl.BlockSpec((D, D), lambda i: (0, 0)),
            pl.BlockSpec((1, D), lambda i: (0, 0)),
        ],
        out_specs=pl.BlockSpec((1024, D), lambda i: (i, 0)),
        out_shape=jax.ShapeDtypeStruct((NPAD, D), jnp.float32),
    )(xp, WlT, b_l)


_mesh = plsc.VectorSubcoreMesh(core_axis_name="c", subcore_axis_name="s")


@functools.partial(
    pl.kernel,
    out_type=(
        jax.ShapeDtypeStruct((2, NPAD, D), jnp.float32),
        jax.ShapeDtypeStruct((2, NPAD), jnp.float32),
    ),
    mesh=_mesh,
    scratch_types=[
        pltpu.VMEM((K,), jnp.int32),
        pltpu.VMEM((K,), jnp.int32),
        pltpu.VMEM((K,), jnp.float32),
        pltpu.VMEM((K, 16), jnp.float32),
        pltpu.VMEM((K, D), jnp.float32),
        pltpu.VMEM((ROWS_PER_SUB,), jnp.float32),
        pltpu.VMEM_SHARED((NPAD, D), jnp.float32),
        pltpu.VMEM_SHARED((NPAD,), jnp.float32),
        pltpu.SemaphoreType.DMA,
    ],
)
def _sc_agg(zn_hbm, src_hbm, dst_hbm, w_hbm, w16_hbm, p_hbm, pw_hbm,
            isrc, idst, wbuf, wbuf16, rows, wz, acc, accw, sem):
    cid = lax.axis_index("c")
    sid = lax.axis_index("s")
    wid = cid * 16 + sid
    zv = jnp.zeros((16,), jnp.float32)

    def _zrow(k, carry):
        for j in range(8):
            rows[k, pl.ds(j * 16, 16)] = zv
        return carry

    lax.fori_loop(0, K, _zrow, 0)

    def _zwz(k, carry):
        wz[pl.ds(k * 16, 16)] = zv
        return carry

    lax.fori_loop(0, ROWS_PER_SUB // 16, _zwz, 0)

    row0 = sid * ROWS_PER_SUB
    for t in range(ROWS_PER_SUB // K):
        pltpu.sync_copy(rows, acc.at[pl.ds(row0 + t * K, K)])
    pltpu.sync_copy(wz, accw.at[pl.ds(row0, ROWS_PER_SUB)])
    plsc.subcore_barrier()

    base = wid * PERW

    def _chunk(c, carry):
        off = base + c * K
        pltpu.sync_copy(src_hbm.at[pl.ds(off, K)], isrc)
        pltpu.sync_copy(dst_hbm.at[pl.ds(off, K)], idst)
        pltpu.sync_copy(w_hbm.at[pl.ds(off, K)], wbuf)
        pltpu.sync_copy(w16_hbm.at[pl.ds(off, K)], wbuf16)
        pltpu.async_copy(zn_hbm.at[isrc], rows, sem).wait()

        def _scale(k, inner):
            wv = wbuf16[k]
            for j in range(8):
                sl = pl.ds(j * 16, 16)
                rows[k, sl] = rows[k, sl] * wv
            return inner

        lax.fori_loop(0, K, _scale, 0)
        pltpu.sync_copy(rows, acc.at[idst], add=True)
        pltpu.sync_copy(wbuf, accw.at[idst], add=True)
        return carry

    lax.fori_loop(0, NCHUNK, _chunk, 0)
    plsc.subcore_barrier()

    pltpu.sync_copy(acc.at[pl.ds(row0, ROWS_PER_SUB)],
                    p_hbm.at[cid, pl.ds(row0, ROWS_PER_SUB)])
    pltpu.sync_copy(accw.at[pl.ds(row0, ROWS_PER_SUB)],
                    pw_hbm.at[cid, pl.ds(row0, ROWS_PER_SUB)])


def _out_body(x_ref, p_ref, pw_ref, wt_ref, b_ref, o_ref):
    aggw = pw_ref[0] + pw_ref[1] + 1.0
    agg = (p_ref[0] + p_ref[1]) / aggw
    h = jnp.dot(x_ref[...], wt_ref[0:D, :], preferred_element_type=jnp.float32)
    h = h + jnp.dot(agg, wt_ref[D:2 * D, :], preferred_element_type=jnp.float32)
    h = jnp.maximum(h + b_ref[...], 0.0)
    nrm = jnp.sqrt(jnp.sum(h * h, axis=1, keepdims=True))
    o_ref[...] = h / jnp.maximum(nrm, 1e-12)


def _out_call(xp, P, Pw_col, WrT, b_r):
    return pl.pallas_call(
        _out_body,
        grid=(NPAD // 1024,),
        in_specs=[
            pl.BlockSpec((1024, D), lambda i: (i, 0)),
            pl.BlockSpec((2, 1024, D), lambda i: (0, i, 0)),
            pl.BlockSpec((2, 1024, 1), lambda i: (0, i, 0)),
            pl.BlockSpec((2 * D, D), lambda i: (0, 0)),
            pl.BlockSpec((1, D), lambda i: (0, 0)),
        ],
        out_specs=pl.BlockSpec((1024, D), lambda i: (i, 0)),
        out_shape=jax.ShapeDtypeStruct((NPAD, D), jnp.float32),
    )(xp, P, Pw_col, WrT, b_r)


def kernel(x, edge_index, w, W_l, b_l, W_r, b_r):
    xp = jnp.pad(x, ((0, NPAD - N), (0, 0)))
    src = jnp.pad(edge_index[0, :], (0, EPAD - E))
    dst = jnp.pad(edge_index[1, :], (0, EPAD - E), constant_values=NPAD - 1)
    wp = jnp.pad(w, (0, EPAD - E))
    w16 = jnp.broadcast_to(wp[:, None], (EPAD, 16))

    zn = _zn_call(xp, W_l.T, b_l.reshape(1, D))
    P, Pw = _sc_agg(zn, src, dst, wp, w16)
    out = _out_call(xp, P, Pw.reshape(2, NPAD, 1), W_r.T, b_r.reshape(1, D))
    return out[:N]
